# baseline (speedup 1.0000x reference)
import functools

import jax
import jax.numpy as jnp
from jax import lax
from jax.experimental import pallas as pl
from jax.experimental.pallas import tpu as pltpu
import jax.experimental.pallas.tpu_sc as plsc

NC = 2
NS = 16
LANES = 16


def _make_spmm_pair(n, f, e, chunk):
    nbuf = 2
    ept = e // NS
    eblk = 800
    nchunks = ept // chunk
    ngroups = nchunks // nbuf
    gpb = eblk // (nbuf * chunk)
    assert nchunks == ngroups * nbuf and eblk == gpb * nbuf * chunk
    assert ept % eblk == 0
    rpt = (n // (8 * NS)) * 8
    rem = n - rpt * NS
    mesh = plsc.VectorSubcoreMesh(core_axis_name="c", subcore_axis_name="s")

    sds = jax.ShapeDtypeStruct((NC * n, f), jnp.float32)

    @functools.partial(
        pl.kernel,
        out_type=(sds, sds),
        mesh=mesh,
        scratch_types=[
            pltpu.VMEM_SHARED((n, f), jnp.float32),
            pltpu.VMEM((eblk,), jnp.int32),
            pltpu.VMEM((eblk,), jnp.int32),
            pltpu.VMEM((eblk,), jnp.float32),
            [pltpu.VMEM((chunk,), jnp.int32) for _ in range(nbuf)],
            [pltpu.VMEM((chunk,), jnp.int32) for _ in range(nbuf)],
            [pltpu.VMEM((chunk, f), jnp.float32) for _ in range(nbuf)],
            [pltpu.VMEM((chunk, f), jnp.float32) for _ in range(nbuf)],
            pltpu.SemaphoreType.DMA,
            [pltpu.SemaphoreType.DMA for _ in range(nbuf)],
            [pltpu.SemaphoreType.DMA for _ in range(nbuf)],
        ],
    )
    def spmm2(y2_hbm, y1_hbm, ci_hbm, ei_hbm, vals_hbm,
              t_hbm, out_hbm,
              acc_sh, col_b, row_b, val_b, idx_v, row_s, rows_v, sc_v,
              esem, gsem, ssem):
        c = lax.axis_index("c")
        s = lax.axis_index("s")
        base0 = s * ept
        coff = c * n
        rb = s * rpt

        def phase(table_hbm, init_hbm, dst_hbm, scale):
            pltpu.sync_copy(init_hbm.at[pl.ds(coff + rb, rpt)],
                            acc_sh.at[pl.ds(rb, rpt)])
            if rem:
                @pl.when(s == NS - 1)
                def _():
                    pltpu.sync_copy(init_hbm.at[pl.ds(coff + rpt * NS, rem)],
                                    acc_sh.at[pl.ds(rpt * NS, rem)])
            plsc.subcore_barrier()

            def fire_gather(b, chunk_idx):
                off = chunk_idx * chunk
                for j in range(chunk // LANES):
                    sl = pl.ds(j * LANES, LANES)
                    idx_v[b][sl] = col_b[pl.ds(off + j * LANES, LANES)] + coff
                pltpu.async_copy(table_hbm.at[idx_v[b]], rows_v[b], gsem[b])

            def group(g, _):
                blk = g // gpb
                lg = g - blk * gpb

                @pl.when(lg == 0)
                def _():
                    eb = base0 + blk * eblk
                    e1 = pltpu.async_copy(ei_hbm.at[pl.ds(e + eb, eblk)], col_b, esem)
                    e2 = pltpu.async_copy(ei_hbm.at[pl.ds(eb, eblk)], row_b, esem)
                    e3 = pltpu.async_copy(vals_hbm.at[pl.ds(eb, eblk)], val_b, esem)
                    e1.wait()
                    e2.wait()
                    e3.wait()
                    for b in range(nbuf):
                        fire_gather(b, b)

                for b in range(nbuf):
                    lc = lg * nbuf + b
                    off = lc * chunk
                    pltpu.make_async_copy(
                        table_hbm.at[idx_v[b]], rows_v[b], gsem[b]).wait()

                    @pl.when(g > 0)
                    def _(b=b):
                        pltpu.make_async_copy(
                            sc_v[b], acc_sh.at[row_s[b]], ssem[b]).wait()

                    for j in range(chunk // LANES):
                        sl = pl.ds(j * LANES, LANES)
                        row_s[b][sl] = row_b[pl.ds(off + j * LANES, LANES)]
                    for gg in range(chunk // LANES):
                        vv = val_b[pl.ds(off + gg * LANES, LANES)] * scale
                        for u in range(LANES):
                            gidx = gg * LANES + u
                            v = vv[u]
                            for j in range(f // LANES):
                                sl = pl.ds(j * LANES, LANES)
                                sc_v[b][gidx, sl] = rows_v[b][gidx, sl] * v
                    pltpu.async_copy(sc_v[b], acc_sh.at[row_s[b]], ssem[b],
                                     add=True)

                    @pl.when(lg < gpb - 1)
                    def _(b=b, lc=lc):
                        fire_gather(b, lc + nbuf)
                return 0

            lax.fori_loop(0, ngroups, group, 0)
            for b in range(nbuf):
                pltpu.make_async_copy(sc_v[b], acc_sh.at[row_s[b]], ssem[b]).wait()
            plsc.subcore_barrier()
            pltpu.sync_copy(acc_sh.at[pl.ds(rb, rpt)],
                            dst_hbm.at[pl.ds(coff + rb, rpt)])
            if rem:
                @pl.when(s == NS - 1)
                def _():
                    pltpu.sync_copy(acc_sh.at[pl.ds(rpt * NS, rem)],
                                    dst_hbm.at[pl.ds(coff + rpt * NS, rem)])
            plsc.subcore_barrier()

        phase(y2_hbm, y1_hbm, t_hbm, 2.0)
        phase(t_hbm, ci_hbm, out_hbm, 1.0)

    return spmm2


def _project(u2d, wp, brows, hp, relu_in, rb):
    m, cin = u2d.shape
    f = 4 * hp
    grid = m // rb

    def body(u_ref, w_ref, b_ref, a_ref, t_ref, c_ref):
        u = u_ref[...]
        if relu_in:
            u = jnp.maximum(u, 0.0)
        y = jnp.dot(u, w_ref[...], preferred_element_type=jnp.float32)
        y = y + b_ref[0:1, :]
        y0 = y[:, 0:hp]
        y1 = y[:, hp:2 * hp]
        y2 = y[:, 2 * hp:3 * hp]
        a_ref[...] = y2
        t_ref[...] = y1
        c_ref[...] = y0 - y2 + b_ref[1:2, 0:hp]

    out_sds = jax.ShapeDtypeStruct((m, hp), jnp.float32)
    return pl.pallas_call(
        body,
        grid=(grid,),
        in_specs=[
            pl.BlockSpec((rb, cin), lambda i: (i, 0)),
            pl.BlockSpec((cin, 128), lambda i: (0, 0)),
            pl.BlockSpec((8, 128), lambda i: (0, 0)),
        ],
        out_specs=[pl.BlockSpec((rb, hp), lambda i: (i, 0))] * 3,
        out_shape=[out_sds] * 3,
    )(u2d, wp, brows)


def _project_bn(o, wp, b8, hp, h, rb, cnt):
    m = o.shape[0]
    grid = m // rb

    def body(o_ref, w_ref, b_ref, a_ref, t_ref, c_ref, acc, ws, cs):
        i = pl.program_id(0)

        @pl.when(i == 0)
        def _():
            acc[...] = jnp.zeros((8, 128), jnp.float32)

        @pl.when(i < grid)
        def _():
            z = jnp.maximum(o_ref[...], 0.0)
            s = jnp.sum(z, axis=0, keepdims=True)
            q = jnp.sum(z * z, axis=0, keepdims=True)
            acc[0:1, :] += s
            acc[1:2, :] += q

        @pl.when(i == grid)
        def _():
            li = lax.broadcasted_iota(jnp.int32, (128, 32), 0)
            di = lax.broadcasted_iota(jnp.int32, (128, 32), 1)
            fm = jnp.where(li % 32 == di, 1.0, 0.0)
            s32 = jnp.dot(acc[0:1, :], fm, preferred_element_type=jnp.float32)
            q32 = jnp.dot(acc[1:2, :], fm, preferred_element_type=jnp.float32)
            mean = s32 / cnt
            var = q32 / cnt - mean * mean
            gam = b_ref[3:4, 0:32]
            bet = b_ref[4:5, 0:32]
            ainv = gam / jnp.sqrt(var + 1e-5)
            c0 = bet - mean * ainv
            ri = lax.broadcasted_iota(jnp.int32, (32, 32), 0)
            ci = lax.broadcasted_iota(jnp.int32, (32, 32), 1)
            dia = jnp.where(ri == ci, 1.0, 0.0) * ainv
            ws[...] = jnp.dot(dia, w_ref[...], preferred_element_type=jnp.float32)
            cs[...] = jnp.concatenate(
                [jnp.dot(c0, w_ref[...], preferred_element_type=jnp.float32),
                 jnp.zeros((7, 128), jnp.float32)], axis=0)

        @pl.when(i >= grid)
        def _():
            z = jnp.maximum(o_ref[...], 0.0)
            c0row = cs[0:1, :]
            cb = b_ref[2:3, 0:hp]
            y2s, y1s, cis = [], [], []
            for b in range(4):
                u = z[:, 32 * b:32 * b + 32]
                y = jnp.dot(u, ws[...], preferred_element_type=jnp.float32)
                y = y + c0row
                y0 = y[:, 0:hp]
                y1 = y[:, hp:2 * hp]
                y2 = y[:, 2 * hp:3 * hp]
                y2s.append(y2)
                y1s.append(y1)
                cis.append(y0 - y2 + cb)
            a_ref[...] = jnp.stack(y2s, axis=1)
            t_ref[...] = jnp.stack(y1s, axis=1)
            c_ref[...] = jnp.stack(cis, axis=1)

    out_sds = jax.ShapeDtypeStruct((m, 4, hp), jnp.float32)
    return pl.pallas_call(
        body,
        grid=(2 * grid,),
        in_specs=[
            pl.BlockSpec((rb, 128), lambda i: (i % grid, 0)),
            pl.BlockSpec((32, 128), lambda i: (0, 0)),
            pl.BlockSpec((8, 128), lambda i: (0, 0)),
        ],
        out_specs=[pl.BlockSpec((rb, 4, hp),
                                lambda i: (jnp.maximum(i - grid, 0), 0, 0))] * 3,
        out_shape=[out_sds] * 3,
        scratch_shapes=[
            pltpu.VMEM((8, 128), jnp.float32),
            pltpu.VMEM((32, 128), jnp.float32),
            pltpu.VMEM((8, 128), jnp.float32),
        ],
    )(o, wp, b8)


def _stats(o, rb):
    m = o.shape[0]
    grid = m // rb

    def body(o_ref, out_ref):
        z = jnp.maximum(o_ref[...], 0.0)
        s = jnp.sum(z, axis=0, keepdims=True)
        q = jnp.sum(z * z, axis=0, keepdims=True)
        out_ref[...] = jnp.concatenate(
            [s, q, jnp.zeros((6, 128), jnp.float32)], axis=0)[None]

    return pl.pallas_call(
        body,
        grid=(grid,),
        in_specs=[pl.BlockSpec((rb, 128), lambda i: (i, 0))],
        out_specs=pl.BlockSpec((1, 8, 128), lambda i: (i, 0, 0)),
        out_shape=jax.ShapeDtypeStruct((grid, 8, 128), jnp.float32),
    )(o)


def _maxpool(o2, f, rb):
    m = o2.shape[0]
    grid = m // rb

    def body(o_ref, out_ref):
        z = jnp.maximum(o_ref[...], 0.0)
        v = jnp.max(z, axis=0, keepdims=True)
        out_ref[...] = jnp.concatenate(
            [v, jnp.zeros((7, f), jnp.float32)], axis=0)[None]

    return pl.pallas_call(
        body,
        grid=(grid,),
        in_specs=[pl.BlockSpec((rb, f), lambda i: (i, 0))],
        out_specs=pl.BlockSpec((1, 8, f), lambda i: (i, 0, 0)),
        out_shape=jax.ShapeDtypeStruct((grid, 8, f), jnp.float32),
    )(o2)


def _log_softmax(lg):
    def body(l_ref, out_ref):
        v = l_ref[...]
        m = jnp.max(v, axis=1, keepdims=True)
        u = v - m
        out_ref[...] = u - jnp.log(jnp.sum(jnp.exp(u), axis=1, keepdims=True))

    return pl.pallas_call(
        body,
        out_shape=jax.ShapeDtypeStruct(lg.shape, jnp.float32),
    )(lg)


def _pad_w(wc, cin, hp, h):
    wp = jnp.zeros((cin, 128), jnp.float32)
    for k in range(3):
        wp = wp.at[:, k * hp:k * hp + h].set(wc[:, k * h:(k + 1) * h])
    return wp


def kernel(x, edge_index, lap_values, W0, b0, g1, be1, W1, b1, g2, be2, W2, b2):
    B, CIN, N = x.shape
    E = edge_index.shape[1]
    K, _, H0 = W0.shape
    H1 = W1.shape[2]
    COUT = W2.shape[2]
    HP2 = 32
    n2 = NC * N

    xt = jnp.transpose(x.reshape(NC, 4, CIN, N), (0, 3, 1, 2)).reshape(n2 * 4, CIN)

    w0c = jnp.transpose(W0, (1, 0, 2)).reshape(CIN, 3 * H0)
    w1c = jnp.transpose(W1, (1, 0, 2)).reshape(H0, 3 * H1)
    w2c = jnp.transpose(W2, (1, 0, 2)).reshape(H1, 3 * COUT)

    spmm_pair = _make_spmm_pair(N, 4 * H0, E, 80)

    def brows_of(c0row3h, convb, hp, h):
        b8 = jnp.zeros((8, 128), jnp.float32)
        for k in range(3):
            b8 = b8.at[0, k * hp:k * hp + h].set(c0row3h[k * h:(k + 1) * h])
        b8 = b8.at[1, 0:h].set(convb)
        return b8

    ei_flat = edge_index.reshape(2 * E)
    cnt = float(B * N)

    def run_spmm(y2, y1, cinit, f):
        _, o = spmm_pair(y2.reshape(n2, f), y1.reshape(n2, f),
                         cinit.reshape(n2, f), ei_flat, lap_values)
        return o

    wp0 = _pad_w(w0c, CIN, H0, H0)
    b80 = brows_of(jnp.zeros((3 * H0,), jnp.float32), b0, H0, H0)
    y2, y1, cinit = _project(xt, wp0, b80, H0, False, 1600)
    o0 = run_spmm(y2, y1, cinit, 4 * H0)

    def bn_layer(o_prev, wc, h, hp, convb, gam, bet):
        wp = _pad_w(wc, wc.shape[0], hp, h)
        b8 = jnp.zeros((8, 128), jnp.float32)
        b8 = b8.at[2, 0:h].set(convb)
        b8 = b8.at[3, 0:32].set(gam)
        b8 = b8.at[4, 0:32].set(bet)
        y2, y1, cinit = _project_bn(o_prev, wp, b8, hp, h, 2000, cnt)
        return run_spmm(y2, y1, cinit, 4 * hp)

    o1 = bn_layer(o0, w1c, H1, H1, b1, g1, be1)
    o2 = bn_layer(o1, w2c, COUT, HP2, b2, g2, be2)

    part = _maxpool(o2, 4 * HP2, 1000)
    nb = part.shape[0]
    mx = jnp.max(part[:, 0, :].reshape(NC, nb // NC, 4 * HP2), axis=1)
    mx = mx.reshape(NC, 4, HP2)
    logits = mx.reshape(B, HP2)[:, :COUT]
    lg = jnp.full((B, 128), -1e30, jnp.float32).at[:, :COUT].set(logits)
    out = _log_softmax(lg)
    return out[:, :COUT]


# speedup vs baseline: 1.0297x; 1.0297x over previous
import functools

import jax
import jax.numpy as jnp
from jax import lax
from jax.experimental import pallas as pl
from jax.experimental.pallas import tpu as pltpu
import jax.experimental.pallas.tpu_sc as plsc

NC = 2
NS = 16
LANES = 16


def _make_spmm_pair(n, f, e, chunk):
    nbuf = 2
    ept = e // NS
    eblk = 800
    nchunks = ept // chunk
    ngroups = nchunks // nbuf
    gpb = eblk // (nbuf * chunk)
    assert nchunks == ngroups * nbuf and eblk == gpb * nbuf * chunk
    assert ept % eblk == 0
    rpt = (n // (8 * NS)) * 8
    rem = n - rpt * NS
    mesh = plsc.VectorSubcoreMesh(core_axis_name="c", subcore_axis_name="s")

    sds = jax.ShapeDtypeStruct((NC * n, f), jnp.float32)

    @functools.partial(
        pl.kernel,
        out_type=(sds, sds),
        mesh=mesh,
        scratch_types=[
            pltpu.VMEM_SHARED((n, f), jnp.float32),
            pltpu.VMEM((eblk,), jnp.int32),
            pltpu.VMEM((eblk,), jnp.int32),
            pltpu.VMEM((eblk,), jnp.float32),
            [pltpu.VMEM((chunk,), jnp.int32) for _ in range(nbuf)],
            [pltpu.VMEM((chunk,), jnp.int32) for _ in range(nbuf)],
            [pltpu.VMEM((chunk, f), jnp.float32) for _ in range(nbuf)],
            [pltpu.VMEM((chunk, f), jnp.float32) for _ in range(nbuf)],
            pltpu.SemaphoreType.DMA,
            [pltpu.SemaphoreType.DMA for _ in range(nbuf)],
            [pltpu.SemaphoreType.DMA for _ in range(nbuf)],
        ],
    )
    def spmm2(y2_hbm, y1_hbm, ci_hbm, ei_hbm, vals_hbm,
              t_hbm, out_hbm,
              acc_sh, col_b, row_b, val_b, idx_v, row_s, rows_v, sc_v,
              esem, gsem, ssem):
        c = lax.axis_index("c")
        s = lax.axis_index("s")
        base0 = s * ept
        coff = c * n
        rb = s * rpt

        def phase(table_hbm, init_hbm, dst_hbm, scale):
            pltpu.sync_copy(init_hbm.at[pl.ds(coff + rb, rpt)],
                            acc_sh.at[pl.ds(rb, rpt)])
            if rem:
                @pl.when(s == NS - 1)
                def _():
                    pltpu.sync_copy(init_hbm.at[pl.ds(coff + rpt * NS, rem)],
                                    acc_sh.at[pl.ds(rpt * NS, rem)])
            plsc.subcore_barrier()

            def fire_gather(b, chunk_idx):
                off = chunk_idx * chunk
                for j in range(chunk // LANES):
                    sl = pl.ds(j * LANES, LANES)
                    idx_v[b][sl] = col_b[pl.ds(off + j * LANES, LANES)] + coff
                pltpu.async_copy(table_hbm.at[idx_v[b]], rows_v[b], gsem[b])

            def group(g, _):
                blk = g // gpb
                lg = g - blk * gpb

                @pl.when(lg == 0)
                def _():
                    eb = base0 + blk * eblk
                    e1 = pltpu.async_copy(ei_hbm.at[pl.ds(e + eb, eblk)], col_b, esem)
                    e2 = pltpu.async_copy(ei_hbm.at[pl.ds(eb, eblk)], row_b, esem)
                    e3 = pltpu.async_copy(vals_hbm.at[pl.ds(eb, eblk)], val_b, esem)
                    e1.wait()
                    e2.wait()
                    e3.wait()
                    for b in range(nbuf):
                        fire_gather(b, b)

                for b in range(nbuf):
                    lc = lg * nbuf + b
                    off = lc * chunk
                    pltpu.make_async_copy(
                        table_hbm.at[idx_v[b]], rows_v[b], gsem[b]).wait()

                    @pl.when(g > 0)
                    def _(b=b):
                        pltpu.make_async_copy(
                            sc_v[b], acc_sh.at[row_s[b]], ssem[b]).wait()

                    for j in range(chunk // LANES):
                        sl = pl.ds(j * LANES, LANES)
                        row_s[b][sl] = row_b[pl.ds(off + j * LANES, LANES)]
                    for gg in range(chunk // LANES):
                        vv = val_b[pl.ds(off + gg * LANES, LANES)] * scale
                        for u in range(LANES):
                            gidx = gg * LANES + u
                            v = vv[u]
                            for j in range(f // LANES):
                                sl = pl.ds(j * LANES, LANES)
                                sc_v[b][gidx, sl] = rows_v[b][gidx, sl] * v
                    pltpu.async_copy(sc_v[b], acc_sh.at[row_s[b]], ssem[b],
                                     add=True)

                    @pl.when(lg < gpb - 1)
                    def _(b=b, lc=lc):
                        fire_gather(b, lc + nbuf)
                return 0

            lax.fori_loop(0, ngroups, group, 0)
            for b in range(nbuf):
                pltpu.make_async_copy(sc_v[b], acc_sh.at[row_s[b]], ssem[b]).wait()
            plsc.subcore_barrier()
            pltpu.sync_copy(acc_sh.at[pl.ds(rb, rpt)],
                            dst_hbm.at[pl.ds(coff + rb, rpt)])
            if rem:
                @pl.when(s == NS - 1)
                def _():
                    pltpu.sync_copy(acc_sh.at[pl.ds(rpt * NS, rem)],
                                    dst_hbm.at[pl.ds(coff + rpt * NS, rem)])
            plsc.subcore_barrier()

        phase(y2_hbm, y1_hbm, t_hbm, 2.0)
        phase(t_hbm, ci_hbm, out_hbm, 1.0)

    return spmm2


def _project(u2d, wp, brows, hp, relu_in, rb):
    m, cin = u2d.shape
    f = 4 * hp
    grid = m // rb

    def body(u_ref, w_ref, b_ref, a_ref, t_ref, c_ref):
        u = u_ref[...]
        if relu_in:
            u = jnp.maximum(u, 0.0)
        y = jnp.dot(u, w_ref[...], preferred_element_type=jnp.float32)
        y = y + b_ref[0:1, :]
        y0 = y[:, 0:hp]
        y1 = y[:, hp:2 * hp]
        y2 = y[:, 2 * hp:3 * hp]
        a_ref[...] = y2
        t_ref[...] = y1
        c_ref[...] = y0 - y2 + b_ref[1:2, 0:hp]

    out_sds = jax.ShapeDtypeStruct((m, hp), jnp.float32)
    return pl.pallas_call(
        body,
        grid=(grid,),
        in_specs=[
            pl.BlockSpec((rb, cin), lambda i: (i, 0)),
            pl.BlockSpec((cin, 128), lambda i: (0, 0)),
            pl.BlockSpec((8, 128), lambda i: (0, 0)),
        ],
        out_specs=[pl.BlockSpec((rb, hp), lambda i: (i, 0))] * 3,
        out_shape=[out_sds] * 3,
    )(u2d, wp, brows)


def _stats(o, rb):
    m = o.shape[0]
    grid = m // rb

    def body(o_ref, out_ref):
        z = jnp.maximum(o_ref[...], 0.0)
        s = jnp.sum(z, axis=0, keepdims=True)
        q = jnp.sum(z * z, axis=0, keepdims=True)
        out_ref[...] = jnp.concatenate(
            [s, q, jnp.zeros((6, 128), jnp.float32)], axis=0)[None]

    return pl.pallas_call(
        body,
        grid=(grid,),
        in_specs=[pl.BlockSpec((rb, 128), lambda i: (i, 0))],
        out_specs=pl.BlockSpec((1, 8, 128), lambda i: (i, 0, 0)),
        out_shape=jax.ShapeDtypeStruct((grid, 8, 128), jnp.float32),
    )(o)


def _maxpool(o2, f, rb):
    m = o2.shape[0]
    grid = m // rb

    def body(o_ref, out_ref):
        z = jnp.maximum(o_ref[...], 0.0)
        v = jnp.max(z, axis=0, keepdims=True)
        out_ref[...] = jnp.concatenate(
            [v, jnp.zeros((7, f), jnp.float32)], axis=0)[None]

    return pl.pallas_call(
        body,
        grid=(grid,),
        in_specs=[pl.BlockSpec((rb, f), lambda i: (i, 0))],
        out_specs=pl.BlockSpec((1, 8, f), lambda i: (i, 0, 0)),
        out_shape=jax.ShapeDtypeStruct((grid, 8, f), jnp.float32),
    )(o2)


def _log_softmax(lg):
    def body(l_ref, out_ref):
        v = l_ref[...]
        m = jnp.max(v, axis=1, keepdims=True)
        u = v - m
        out_ref[...] = u - jnp.log(jnp.sum(jnp.exp(u), axis=1, keepdims=True))

    return pl.pallas_call(
        body,
        out_shape=jax.ShapeDtypeStruct(lg.shape, jnp.float32),
    )(lg)


def _pad_w(wc, cin, hp, h):
    wp = jnp.zeros((cin, 128), jnp.float32)
    for k in range(3):
        wp = wp.at[:, k * hp:k * hp + h].set(wc[:, k * h:(k + 1) * h])
    return wp


def kernel(x, edge_index, lap_values, W0, b0, g1, be1, W1, b1, g2, be2, W2, b2):
    B, CIN, N = x.shape
    E = edge_index.shape[1]
    K, _, H0 = W0.shape
    H1 = W1.shape[2]
    COUT = W2.shape[2]
    HP2 = 32
    n2 = NC * N

    xt = jnp.transpose(x.reshape(NC, 4, CIN, N), (0, 3, 1, 2)).reshape(n2 * 4, CIN)

    w0c = jnp.transpose(W0, (1, 0, 2)).reshape(CIN, 3 * H0)
    w1c = jnp.transpose(W1, (1, 0, 2)).reshape(H0, 3 * H1)
    w2c = jnp.transpose(W2, (1, 0, 2)).reshape(H1, 3 * COUT)

    spmm_pair = _make_spmm_pair(N, 4 * H0, E, 80)

    def brows_of(c0row3h, convb, hp, h):
        b8 = jnp.zeros((8, 128), jnp.float32)
        for k in range(3):
            b8 = b8.at[0, k * hp:k * hp + h].set(c0row3h[k * h:(k + 1) * h])
        b8 = b8.at[1, 0:h].set(convb)
        return b8

    def cheb_layer(u2d, wc, cin, h, hp, convb, bn, relu_in):
        wcc = wc
        c0row = jnp.zeros((3 * h,), jnp.float32)
        if bn is not None:
            a, c0 = bn
            wcc = wc * a[:, None]
            c0row = c0 @ wc
        wp = _pad_w(wcc, cin, hp, h)
        b8 = brows_of(c0row, convb, hp, h)
        y2, y1, cinit = _project(u2d, wp, b8, hp, relu_in, 1600)
        f = 4 * hp
        _, o = spmm_pair(y2.reshape(n2, f), y1.reshape(n2, f),
                         cinit.reshape(n2, f), edge_index.reshape(2 * E),
                         lap_values)
        return o

    o0 = cheb_layer(xt, w0c, CIN, H0, H0, b0, None, False)

    p0 = _stats(o0, 1000)
    s0 = jnp.sum(p0[:, 0, :], axis=0).reshape(4, H0).sum(axis=0)
    q0 = jnp.sum(p0[:, 1, :], axis=0).reshape(4, H0).sum(axis=0)
    cnt = float(B * N)
    m0 = s0 / cnt
    v0 = q0 / cnt - m0 * m0
    a0 = g1 / jnp.sqrt(v0 + 1e-5)
    c00 = be1 - m0 * a0

    o1 = cheb_layer(o0.reshape(n2 * 4, H0), w1c, H0, H1, H1, b1,
                    (a0, c00), True)

    p1 = _stats(o1, 1000)
    s1 = jnp.sum(p1[:, 0, :], axis=0).reshape(4, H1).sum(axis=0)
    q1 = jnp.sum(p1[:, 1, :], axis=0).reshape(4, H1).sum(axis=0)
    m1 = s1 / cnt
    v1 = q1 / cnt - m1 * m1
    a1 = g2 / jnp.sqrt(v1 + 1e-5)
    c01 = be2 - m1 * a1

    o2 = cheb_layer(o1.reshape(n2 * 4, H1), w2c, H1, COUT, HP2, b2,
                    (a1, c01), True)

    part = _maxpool(o2, 4 * HP2, 1000)
    nb = part.shape[0]
    mx = jnp.max(part[:, 0, :].reshape(NC, nb // NC, 4 * HP2), axis=1)
    mx = mx.reshape(NC, 4, HP2)
    logits = mx.reshape(B, HP2)[:, :COUT]
    lg = jnp.full((B, 128), -1e30, jnp.float32).at[:, :COUT].set(logits)
    out = _log_softmax(lg)
    return out[:, :COUT]
